# baseline (speedup 1.0000x reference)
import functools

import jax
import jax.numpy as jnp
from jax import lax
from jax.experimental import pallas as pl
from jax.experimental.pallas import tpu as pltpu
from jax.experimental.pallas import tpu_sc as plsc

B, C, H, W = 8, 384, 224, 224
P = 200
HW = H * W
CHW = C * HW
NPTS = 2 * B * P
NW = 32
PPW = NPTS // NW
ROWS_PER_PT = C // 128
NROWS = PPW * ROWS_PER_PT
NROWS_PAD = 304
EPW_PAD = NROWS_PAD * 128
LANES = 16


def _body(img_hbm, xs_hbm, ys_hbm, out_hbm, xs_v, ys_v, base_v, idx_v,
          gat_v, sem):
    wid = lax.axis_index("s") * 2 + lax.axis_index("c")
    b = (wid % 16) // 2

    pltpu.sync_copy(xs_hbm.at[wid], xs_v)
    pltpu.sync_copy(ys_hbm.at[wid], ys_v)

    iota = lax.iota(jnp.int32, LANES)

    for k in range(128 // LANES):
        xv = xs_v[pl.ds(k * LANES, LANES)]
        yv = ys_v[pl.ds(k * LANES, LANES)]
        base_v[pl.ds(k * LANES, LANES)] = b * CHW + yv * W + xv

    def fill(p, carry):
        bp = base_v[pl.ds(p, LANES)][0]
        for cb in range(C // LANES):
            idx_v[pl.ds(p * C + cb * LANES, LANES)] = (
                bp + (cb * LANES + iota) * HW)
        return carry

    lax.fori_loop(0, PPW, fill, 0)

    zeros = iota * 0
    for q in range(NROWS * 128, EPW_PAD, LANES):
        idx_v[pl.ds(q, LANES)] = zeros

    pltpu.async_copy(img_hbm.at[idx_v], gat_v, sem).wait()

    pltpu.sync_copy(gat_v, out_hbm.at[pl.ds(wid * EPW_PAD, EPW_PAD)])


@jax.jit
def _crop_gather(img_flat, xs_pad, ys_pad):
    kern = functools.partial(
        pl.kernel,
        out_type=jax.ShapeDtypeStruct((NW * EPW_PAD,), jnp.float32),
        mesh=plsc.VectorSubcoreMesh(core_axis_name="c",
                                    subcore_axis_name="s"),
        scratch_types=[
            pltpu.VMEM((128,), jnp.int32),
            pltpu.VMEM((128,), jnp.int32),
            pltpu.VMEM((128,), jnp.int32),
            pltpu.VMEM((EPW_PAD,), jnp.int32),
            pltpu.VMEM((EPW_PAD,), jnp.float32),
            pltpu.SemaphoreType.DMA,
        ],
    )(_body)
    return kern(img_flat, xs_pad, ys_pad)


def kernel(imgs, batch_cpoints, batch_npoints):
    img_flat = imgs.reshape(-1)
    pts = jnp.concatenate(
        [batch_cpoints.reshape(-1, 2), batch_npoints.reshape(-1, 2)], axis=0)
    xs = pts[:, 0].reshape(NW, PPW)
    ys = pts[:, 1].reshape(NW, PPW)
    pad = ((0, 0), (0, 128 - PPW))
    xs_pad = jnp.pad(xs, pad)
    ys_pad = jnp.pad(ys, pad)
    out = _crop_gather(img_flat, xs_pad, ys_pad)
    batch_crop_imgs = (out.reshape(NW, EPW_PAD)[:, :PPW * C]
                       .reshape(NPTS, C))
    return (batch_crop_imgs, NPTS // 2, NPTS)


# speedup vs baseline: 52.1230x; 52.1230x over previous
import functools

import jax
import jax.numpy as jnp
from jax import lax
from jax.experimental import pallas as pl
from jax.experimental.pallas import tpu as pltpu
from jax.experimental.pallas import tpu_sc as plsc

B, C, H, W = 8, 384, 224, 224
P = 200
NPTS = 2 * B * P
NWORK = 25
PPW = NPTS // NWORK
LANES = 16


def _body(tbl_hbm, xs_hbm, ys_hbm, out_hbm, xs_v, ys_v, idx_v, rows_v, sem):
    wid = lax.axis_index("s") * 2 + lax.axis_index("c")

    @pl.when(wid < NWORK)
    def _():
        start = wid * PPW
        pltpu.sync_copy(xs_hbm.at[pl.ds(start, PPW)], xs_v)
        pltpu.sync_copy(ys_hbm.at[pl.ds(start, PPW)], ys_v)

        iota = lax.iota(jnp.int32, LANES)
        for k in range(PPW // LANES):
            xv = xs_v[pl.ds(k * LANES, LANES)]
            yv = ys_v[pl.ds(k * LANES, LANES)]
            gv = start + k * LANES + iota
            bv = lax.div(lax.rem(gv, jnp.int32(B * P)), jnp.int32(P))
            idx_v[pl.ds(k * LANES, LANES)] = (bv * H + yv) * W + xv

        pltpu.async_copy(tbl_hbm.at[idx_v], rows_v, sem).wait()
        pltpu.sync_copy(rows_v, out_hbm.at[pl.ds(wid * PPW, PPW)])


@jax.jit
def _crop_gather(tbl, xs, ys):
    kern = functools.partial(
        pl.kernel,
        out_type=jax.ShapeDtypeStruct((NPTS, C), jnp.float32),
        mesh=plsc.VectorSubcoreMesh(core_axis_name="c",
                                    subcore_axis_name="s"),
        scratch_types=[
            pltpu.VMEM((PPW,), jnp.int32),
            pltpu.VMEM((PPW,), jnp.int32),
            pltpu.VMEM((PPW,), jnp.int32),
            pltpu.VMEM((PPW, C), jnp.float32),
            pltpu.SemaphoreType.DMA,
        ],
    )(_body)
    return kern(tbl, xs, ys)


def kernel(imgs, batch_cpoints, batch_npoints):
    tbl = imgs.transpose(0, 2, 3, 1).reshape(B * H * W, C)
    pts = jnp.concatenate(
        [batch_cpoints.reshape(-1, 2), batch_npoints.reshape(-1, 2)], axis=0)
    batch_crop_imgs = _crop_gather(tbl, pts[:, 0], pts[:, 1])
    return (batch_crop_imgs, NPTS // 2, NPTS)


# speedup vs baseline: 53.8536x; 1.0332x over previous
import functools

import jax
import jax.numpy as jnp
from jax import lax
from jax.experimental import pallas as pl
from jax.experimental.pallas import tpu as pltpu
from jax.experimental.pallas import tpu_sc as plsc

B, C, H, W = 8, 384, 224, 224
P = 200
NPTS = 2 * B * P
CHUNK = 104
LANES = 16


def _body(tbl_hbm, cpt_hbm, npt_hbm, out_hbm, pts_v, idx_v, rows_v, sem):
    wid = lax.axis_index("s") * 2 + lax.axis_index("c")
    seg = wid // 2
    arr = seg // 8
    b = seg % 8
    off = (wid % 2) * 96

    def run(pts_ref):
        pltpu.sync_copy(pts_ref.at[b], pts_v)
        for s in list(range(0, P - LANES, LANES)) + [P - LANES]:
            xv = pts_v[0, pl.ds(s, LANES)]
            yv = pts_v[1, pl.ds(s, LANES)]
            idx_v[pl.ds(s, LANES)] = (b * H + yv) * W + xv
        pltpu.async_copy(tbl_hbm.at[idx_v.at[pl.ds(off, CHUNK)]], rows_v,
                         sem).wait()
        base = arr * (B * P) + b * P + off
        pltpu.sync_copy(rows_v, out_hbm.at[pl.ds(base, CHUNK)])

    @pl.when(arr == 0)
    def _():
        run(cpt_hbm)

    @pl.when(arr == 1)
    def _():
        run(npt_hbm)


@jax.jit
def _crop_gather(tbl, cpt, npt):
    kern = functools.partial(
        pl.kernel,
        out_type=jax.ShapeDtypeStruct((NPTS, C), jnp.float32),
        mesh=plsc.VectorSubcoreMesh(core_axis_name="c",
                                    subcore_axis_name="s"),
        scratch_types=[
            pltpu.VMEM((2, P), jnp.int32),
            pltpu.VMEM((P, ), jnp.int32),
            pltpu.VMEM((CHUNK, C), jnp.float32),
            pltpu.SemaphoreType.DMA,
        ],
    )(_body)
    return kern(tbl, cpt, npt)


def kernel(imgs, batch_cpoints, batch_npoints):
    tbl = imgs.transpose(0, 2, 3, 1).reshape(B * H * W, C)
    cpt = batch_cpoints.transpose(0, 2, 1)
    npt = batch_npoints.transpose(0, 2, 1)
    batch_crop_imgs = _crop_gather(tbl, cpt, npt)
    return (batch_crop_imgs, NPTS // 2, NPTS)
